# baseline (speedup 1.0000x reference)
import functools

import jax
import jax.numpy as jnp
from jax import lax
from jax.experimental import pallas as pl
from jax.experimental.pallas import tpu as pltpu
from jax.experimental.pallas import tpu_sc as plsc

B, L, H = 1024, 200, 128
N = B * L
NC, NS = 2, 16
NW = NC * NS
TPW = N // NW
T = 128
NCHUNK = TPW // T
NG = T // 16
UN = 4
HC = H // 16
NRESP, NELAP, NPOS = 4, 302, 256
EPS = 1e-12


def _rsqrt(x):
    z = x * jnp.float32(2.0 ** 64)
    r = jnp.full((16,), 2.0 ** 32, jnp.float32)
    for k in (64, 32, 16, 8, 4, 2):
        big = z >= jnp.float32(2.0 ** k)
        z = jnp.where(big, z * jnp.float32(2.0 ** -k), z)
        r = jnp.where(big, r * jnp.float32(2.0 ** (-k / 2)), r)
    y = jnp.float32(7.0 / 6.0) - z * jnp.float32(1.0 / 6.0)
    for _ in range(4):
        y = y * (1.5 - 0.5 * z * y * y)
    return y * r


def _sc_body(ids_hbm, rt_hbm, et_hbm, lt_hbm, pt_hbm, g_hbm, b_hbm, out_hbm,
             idx_v, rtab_v, etab_v, ptab_v, lr_v, emb_v, out_v, gb_v, sem):
    wid = lax.axis_index("s") * NC + lax.axis_index("c")
    base = wid * TPW

    pltpu.sync_copy(rt_hbm, rtab_v)
    pltpu.sync_copy(et_hbm, etab_v)
    pltpu.sync_copy(pt_hbm.at[pl.ds(0, NPOS)], ptab_v)
    pltpu.sync_copy(g_hbm, gb_v.at[0])
    pltpu.sync_copy(b_hbm, gb_v.at[1])
    gamma = [gb_v[0, pl.ds(j * 16, 16)] for j in range(HC)]
    beta = [gb_v[1, pl.ds(j * 16, 16)] for j in range(HC)]

    lanes = lax.iota(jnp.int32, 16)
    zero16 = jnp.zeros((16,), jnp.float32)

    def chunk_body(c, carry):
        cb = base + c * T
        pltpu.sync_copy(ids_hbm.at[:, pl.ds(cb, T)], idx_v)
        pltpu.async_copy(lt_hbm.at[idx_v.at[2]], lr_v, sem).wait()

        for g in range(NG):
            g16 = g * 16
            idr = idx_v[0, pl.ds(g16, 16)]
            ide = idx_v[1, pl.ds(g16, 16)]
            idp = idx_v[3, pl.ds(g16, 16)]
            rowsl = lanes + g16

            def p1_body(i, acc):
                a = list(acc)
                for k in range(UN):
                    h = i * UN + k
                    hm = h & 15
                    col = ((lanes + hm) & 15) + (h - hm)
                    s = (plsc.load_gather(rtab_v, [idr, col])
                         + plsc.load_gather(etab_v, [ide, col])
                         + plsc.load_gather(ptab_v, [idp, col])
                         + plsc.load_gather(lr_v, [rowsl, col]))
                    plsc.store_scatter(emb_v, [lanes, col], s)
                    a[k] = a[k] + s
                    a[UN + k] = a[UN + k] + s * s
                return tuple(a)

            acc = lax.fori_loop(0, H // UN, p1_body,
                                (zero16,) * (2 * UN), unroll=False)
            mean = (acc[0] + acc[1] + acc[2] + acc[3]) * (1.0 / H)
            msq = (acc[4] + acc[5] + acc[6] + acc[7]) * (1.0 / H)
            var = msq - mean * mean
            rs = _rsqrt(jnp.maximum(var, 0.0) + EPS)

            def p2_body(t, carry2):
                tloc = jnp.full((16,), t, jnp.int32)
                mean_s = mean.at[tloc].get(mode="promise_in_bounds")
                rs_s = rs.at[tloc].get(mode="promise_in_bounds")
                for j in range(HC):
                    e = emb_v[t, pl.ds(j * 16, 16)]
                    out_v[g16 + t, pl.ds(j * 16, 16)] = (
                        (e - mean_s) * (rs_s * gamma[j]) + beta[j])
                return carry2

            lax.fori_loop(0, 16, p2_body, 0, unroll=False)

        pltpu.sync_copy(out_v, out_hbm.at[pl.ds(cb, T)])
        return carry

    lax.fori_loop(0, NCHUNK, chunk_body, 0, unroll=False)


@jax.jit
def _pickt_sc(ids, rt, et, lt, ptab, gamma, beta):
    mesh = plsc.VectorSubcoreMesh(core_axis_name="c", subcore_axis_name="s")
    f = functools.partial(
        pl.kernel,
        out_type=jax.ShapeDtypeStruct((N, H), jnp.float32),
        mesh=mesh,
        scratch_types=[
            pltpu.VMEM((4, T), jnp.int32),
            pltpu.VMEM((NRESP, H), jnp.float32),
            pltpu.VMEM((NELAP, H), jnp.float32),
            pltpu.VMEM((NPOS, H), jnp.float32),
            pltpu.VMEM((T, H), jnp.float32),
            pltpu.VMEM((16, H), jnp.float32),
            pltpu.VMEM((T, H), jnp.float32),
            pltpu.VMEM((2, H), jnp.float32),
            pltpu.SemaphoreType.DMA,
        ],
        compiler_params=pltpu.CompilerParams(needs_layout_passes=False),
    )(_sc_body)
    return f(ids, rt, et, lt, ptab, gamma, beta)


def kernel(response_ids, elapsed_ids, lag_ids, position_ids,
           response_table, elapsed_table, lag_table, position_table,
           ln_gamma, ln_beta):
    ids = jnp.stack([
        response_ids.reshape(-1).astype(jnp.int32),
        elapsed_ids.reshape(-1).astype(jnp.int32),
        lag_ids.reshape(-1).astype(jnp.int32),
        position_ids.reshape(-1).astype(jnp.int32),
    ])
    out = _pickt_sc(ids, response_table, elapsed_table, lag_table,
                    position_table, ln_gamma, ln_beta)
    return out.reshape(B, L, H)
